# Initial kernel scaffold; baseline (speedup 1.0000x reference)
import functools

import jax
import jax.numpy as jnp
import numpy as np
from jax import lax
from jax.experimental import pallas as pl
from jax.experimental.pallas import tpu as pltpu
from jax.experimental.pallas import tpu_sc as plsc

N = 10000
E = 320000
D = 128
BN_EPS = 1e-5

NC = 2
NS = 16
NW = NC * NS

EPT = E // NW
K = 80
CHUNKS = EPT // K
ZR = 125
RPT = N // NS

_mesh = plsc.VectorSubcoreMesh(core_axis_name="c", subcore_axis_name="s")


@functools.partial(
    pl.kernel,
    out_type=jax.ShapeDtypeStruct((NC, N, 16), jnp.float32),
    mesh=_mesh,
    scratch_types=[
        pltpu.VMEM((CHUNKS, K), jnp.int32),
        pltpu.VMEM((K, 16), jnp.float32),
        pltpu.VMEM((ZR, 16), jnp.float32),
        pltpu.VMEM_SHARED((N, 16), jnp.float32),
    ],
)
def _deg_sc(dst_hbm, out_hbm, dst_v, ones_v, zbuf, acc):
    cid = lax.axis_index("c")
    sid = lax.axis_index("s")
    wid = sid * NC + cid

    def fill(r, carry):
        zbuf[r, :] = jnp.zeros((16,), jnp.float32)
        return carry

    lax.fori_loop(0, ZR, fill, 0)

    def fill1(r, carry):
        ones_v[r, :] = jnp.ones((16,), jnp.float32)
        return carry

    lax.fori_loop(0, K, fill1, 0)

    for z in range(RPT // ZR):
        pltpu.sync_copy(zbuf, acc.at[pl.ds(sid * RPT + z * ZR, ZR)])
    plsc.subcore_barrier()

    pltpu.sync_copy(dst_hbm.at[pl.ds(wid * CHUNKS, CHUNKS)], dst_v)

    def body(c, carry):
        pltpu.sync_copy(ones_v, acc.at[dst_v.at[c]], add=True)
        return carry

    lax.fori_loop(0, CHUNKS, body, 0)

    plsc.subcore_barrier()
    pltpu.sync_copy(acc.at[pl.ds(sid * RPT, RPT)],
                    out_hbm.at[cid, pl.ds(sid * RPT, RPT)])


@functools.partial(
    pl.kernel,
    out_type=jax.ShapeDtypeStruct((NC, N, D), jnp.float32),
    mesh=_mesh,
    scratch_types=[
        pltpu.VMEM((CHUNKS, K), jnp.int32),
        pltpu.VMEM((CHUNKS, K), jnp.int32),
        pltpu.VMEM((2, K, D), jnp.float32),
        pltpu.VMEM((ZR, D), jnp.float32),
        pltpu.VMEM_SHARED((N, D), jnp.float32),
        pltpu.SemaphoreType.DMA,
    ],
)
def _hop_sc(g_hbm, src_hbm, dst_hbm, out_hbm, src_v, dst_v, rows_v, zbuf, acc, sem):
    cid = lax.axis_index("c")
    sid = lax.axis_index("s")
    wid = sid * NC + cid

    def fill(r, carry):
        for j in range(D // 16):
            zbuf[r, pl.ds(j * 16, 16)] = jnp.zeros((16,), jnp.float32)
        return carry

    lax.fori_loop(0, ZR, fill, 0)
    for z in range(RPT // ZR):
        pltpu.sync_copy(zbuf, acc.at[pl.ds(sid * RPT + z * ZR, ZR)])
    plsc.subcore_barrier()

    pltpu.sync_copy(src_hbm.at[pl.ds(wid * CHUNKS, CHUNKS)], src_v)
    pltpu.sync_copy(dst_hbm.at[pl.ds(wid * CHUNKS, CHUNKS)], dst_v)

    def body(c, carry):
        pltpu.async_copy(g_hbm.at[src_v.at[c]], rows_v.at[0], sem).wait()
        pltpu.sync_copy(rows_v.at[0], acc.at[dst_v.at[c]], add=True)
        return carry

    lax.fori_loop(0, CHUNKS, body, 0)

    plsc.subcore_barrier()
    pltpu.sync_copy(acc.at[pl.ds(sid * RPT, RPT)],
                    out_hbm.at[cid, pl.ds(sid * RPT, RPT)])


_BR = 1000


def _prep_body(dp_ref, x_ref, o_ref):
    deg = dp_ref[0, :, :1] + dp_ref[1, :, :1] + 1.0
    o_ref[...] = x_ref[...] * lax.rsqrt(deg)


def _comb_body(dp_ref, g_ref, p_ref, o_ref):
    deg = dp_ref[0, :, :1] + dp_ref[1, :, :1] + 1.0
    o_ref[...] = (p_ref[0] + p_ref[1] + g_ref[...]) / deg


def _final_body(dp_ref, g_ref, p_ref, w1_ref, b1_ref, gm_ref, bt_ref,
                w2_ref, b2_ref, o_ref):
    deg = dp_ref[0, :, :1] + dp_ref[1, :, :1] + 1.0
    h2 = (p_ref[0] + p_ref[1] + g_ref[...]) * lax.rsqrt(deg)
    t = lax.dot_general(h2, w1_ref[...], (((1,), (1,)), ((), ())),
                        preferred_element_type=jnp.float32)
    bn_scale = np.float32(1.0 / np.sqrt(1.0 + BN_EPS))
    t = (t + b1_ref[...]) * (gm_ref[...] * bn_scale) + bt_ref[...]
    t = jnp.maximum(t, 0.0)
    o_ref[...] = lax.dot_general(t, w2_ref[...], (((1,), (1,)), ((), ())),
                                 preferred_element_type=jnp.float32) + b2_ref[...]


def _dp_spec():
    return pl.BlockSpec((NC, _BR, 16), lambda i: (0, i, 0))


def _row_spec():
    return pl.BlockSpec((_BR, D), lambda i: (i, 0))


def _p_spec():
    return pl.BlockSpec((NC, _BR, D), lambda i: (0, i, 0))


def _full_spec(shape):
    nd = len(shape)
    return pl.BlockSpec(shape, lambda i, _nd=nd: (0,) * _nd)


_prep_tc = pl.pallas_call(
    _prep_body,
    grid=(N // _BR,),
    in_specs=[_dp_spec(), _row_spec()],
    out_specs=_row_spec(),
    out_shape=jax.ShapeDtypeStruct((N, D), jnp.float32),
)

_comb_tc = pl.pallas_call(
    _comb_body,
    grid=(N // _BR,),
    in_specs=[_dp_spec(), _row_spec(), _p_spec()],
    out_specs=_row_spec(),
    out_shape=jax.ShapeDtypeStruct((N, D), jnp.float32),
)

_final_tc = pl.pallas_call(
    _final_body,
    grid=(N // _BR,),
    in_specs=[_dp_spec(), _row_spec(), _p_spec(),
              _full_spec((D, D)), _full_spec((1, D)), _full_spec((1, D)),
              _full_spec((1, D)), _full_spec((D, D)), _full_spec((1, D))],
    out_specs=_row_spec(),
    out_shape=jax.ShapeDtypeStruct((N, D), jnp.float32),
)


def kernel(x, edge_index, W1, b1, gamma, beta, W2, b2):
    src = edge_index[0].astype(jnp.int32).reshape(E // K, K)
    dst = edge_index[1].astype(jnp.int32).reshape(E // K, K)
    b1r = b1.reshape(1, D)
    gmr = gamma.reshape(1, D)
    btr = beta.reshape(1, D)
    b2r = b2.reshape(1, D)

    degp = _deg_sc(dst)
    g0 = _prep_tc(degp, x)
    p = _hop_sc(g0, src, dst)
    g1 = _comb_tc(degp, g0, p)
    p2 = _hop_sc(g1, src, dst)
    out = _final_tc(degp, g1, p2, W1, b1r, gmr, btr, W2, b2r)
    return out


# speedup vs baseline: 8.8244x; 8.8244x over previous
import functools

import jax
import jax.numpy as jnp
import numpy as np
from jax import lax
from jax.experimental import pallas as pl
from jax.experimental.pallas import tpu as pltpu
from jax.experimental.pallas import tpu_sc as plsc

N = 10000
NP = 10240
E = 320000
D = 128
DH = D // 2
BN_EPS = 1e-5

NC = 2
NS = 16
NW = NC * NS

K = 80
ZR = 128
SLAB = NP // NS

EPT_DEG = E // NW
CH_DEG = EPT_DEG // K
EPT_HOP = E // NS
CH_HOP = EPT_HOP // K

_mesh = plsc.VectorSubcoreMesh(core_axis_name="c", subcore_axis_name="s")
_sc_params = pltpu.CompilerParams(use_tc_tiling_on_sc=False)


@functools.partial(
    pl.kernel,
    out_type=jax.ShapeDtypeStruct((NC, NP, 16), jnp.float32),
    mesh=_mesh,
    scratch_types=[
        pltpu.VMEM((CH_DEG, K), jnp.int32),
        pltpu.VMEM((K, 16), jnp.float32),
        pltpu.VMEM((ZR, 16), jnp.float32),
        pltpu.VMEM_SHARED((NP, 16), jnp.float32),
    ],
    compiler_params=_sc_params,
)
def _deg_sc(dst_hbm, out_hbm, dst_v, ones_v, zbuf, acc):
    cid = lax.axis_index("c")
    sid = lax.axis_index("s")
    wid = sid * NC + cid

    def fill(r, carry):
        zbuf[r, :] = jnp.zeros((16,), jnp.float32)
        return carry

    lax.fori_loop(0, ZR, fill, 0)

    def fill1(r, carry):
        ones_v[r, :] = jnp.ones((16,), jnp.float32)
        return carry

    lax.fori_loop(0, K, fill1, 0)

    for z in range(SLAB // ZR):
        pltpu.sync_copy(zbuf, acc.at[pl.ds(sid * SLAB + z * ZR, ZR)])
    plsc.subcore_barrier()

    def body(c, carry):
        pltpu.sync_copy(dst_hbm.at[pl.ds(wid * EPT_DEG + c * K, K)], dst_v.at[c])
        pltpu.sync_copy(ones_v, acc.at[dst_v.at[c]], add=True)
        return carry

    lax.fori_loop(0, CH_DEG, body, 0)

    plsc.subcore_barrier()
    pltpu.sync_copy(acc.at[pl.ds(sid * SLAB, SLAB)],
                    out_hbm.at[cid, pl.ds(sid * SLAB, SLAB)])


@functools.partial(
    pl.kernel,
    out_type=jax.ShapeDtypeStruct((NC, NP, DH), jnp.float32),
    mesh=_mesh,
    scratch_types=[
        pltpu.VMEM((CH_HOP, K), jnp.int32),
        pltpu.VMEM((CH_HOP, K), jnp.int32),
        pltpu.VMEM((2, K, DH), jnp.float32),
        pltpu.VMEM((ZR, DH), jnp.float32),
        pltpu.VMEM_SHARED((NP, DH), jnp.float32),
        pltpu.SemaphoreType.DMA,
    ],
    compiler_params=_sc_params,
)
def _hop_sc(g_hbm, src_hbm, dst_hbm, out_hbm, src_v, dst_v, rows_v, zbuf, acc, sem):
    cid = lax.axis_index("c")
    sid = lax.axis_index("s")

    def fill(r, carry):
        for j in range(DH // 16):
            zbuf[r, pl.ds(j * 16, 16)] = jnp.zeros((16,), jnp.float32)
        return carry

    lax.fori_loop(0, ZR, fill, 0)
    for z in range(SLAB // ZR):
        pltpu.sync_copy(zbuf, acc.at[pl.ds(sid * SLAB + z * ZR, ZR)])
    plsc.subcore_barrier()

    off = cid * N

    def body(c, carry):
        pltpu.sync_copy(src_hbm.at[pl.ds(sid * EPT_HOP + c * K, K)], src_v.at[c])
        pltpu.sync_copy(dst_hbm.at[pl.ds(sid * EPT_HOP + c * K, K)], dst_v.at[c])
        for j in range(K // 16):
            sl = pl.ds(j * 16, 16)
            src_v[c, sl] = src_v[c, sl] + off
        pltpu.async_copy(g_hbm.at[src_v.at[c]], rows_v.at[0], sem).wait()
        pltpu.sync_copy(rows_v.at[0], acc.at[dst_v.at[c]], add=True)
        return carry

    lax.fori_loop(0, CH_HOP, body, 0)

    plsc.subcore_barrier()
    pltpu.sync_copy(acc.at[pl.ds(sid * SLAB, SLAB)],
                    out_hbm.at[cid, pl.ds(sid * SLAB, SLAB)])


_BR = 1000


def _prep_body(dp_ref, x_ref, o_ref):
    deg = dp_ref[0, :, :1] + dp_ref[1, :, :1] + 1.0
    g = x_ref[...] * lax.rsqrt(deg)
    o_ref[0] = g[:, :DH]
    o_ref[1] = g[:, DH:]


def _comb_body(dp_ref, g_ref, p_ref, o_ref):
    deg = dp_ref[0, :, :1] + dp_ref[1, :, :1] + 1.0
    o_ref[0] = (p_ref[0] + g_ref[0]) / deg
    o_ref[1] = (p_ref[1] + g_ref[1]) / deg


def _final_body(dp_ref, g_ref, p_ref, w1_ref, b1_ref, gm_ref, bt_ref,
                w2_ref, b2_ref, o_ref):
    deg = dp_ref[0, :, :1] + dp_ref[1, :, :1] + 1.0
    h2 = jnp.concatenate([p_ref[0] + g_ref[0], p_ref[1] + g_ref[1]], axis=1)
    h2 = h2 * lax.rsqrt(deg)
    t = lax.dot_general(h2, w1_ref[...], (((1,), (1,)), ((), ())),
                        preferred_element_type=jnp.float32)
    bn_scale = np.float32(1.0 / np.sqrt(1.0 + BN_EPS))
    t = (t + b1_ref[...]) * (gm_ref[...] * bn_scale) + bt_ref[...]
    t = jnp.maximum(t, 0.0)
    o_ref[...] = lax.dot_general(t, w2_ref[...], (((1,), (1,)), ((), ())),
                                 preferred_element_type=jnp.float32) + b2_ref[...]


def _dp_spec():
    return pl.BlockSpec((NC, _BR, 16), lambda i: (0, i, 0))


def _x_spec():
    return pl.BlockSpec((_BR, D), lambda i: (i, 0))


def _g_spec():
    return pl.BlockSpec((2, _BR, DH), lambda i: (0, i, 0))


def _p_spec():
    return pl.BlockSpec((NC, _BR, DH), lambda i: (0, i, 0))


def _full_spec(shape):
    nd = len(shape)
    return pl.BlockSpec(shape, lambda i, _nd=nd: (0,) * _nd)


_prep_tc = pl.pallas_call(
    _prep_body,
    grid=(N // _BR,),
    in_specs=[_dp_spec(), _x_spec()],
    out_specs=_g_spec(),
    out_shape=jax.ShapeDtypeStruct((2, N, DH), jnp.float32),
)

_comb_tc = pl.pallas_call(
    _comb_body,
    grid=(N // _BR,),
    in_specs=[_dp_spec(), _g_spec(), _p_spec()],
    out_specs=_g_spec(),
    out_shape=jax.ShapeDtypeStruct((2, N, DH), jnp.float32),
)

_final_tc = pl.pallas_call(
    _final_body,
    grid=(N // _BR,),
    in_specs=[_dp_spec(), _g_spec(), _p_spec(),
              _full_spec((D, D)), _full_spec((1, D)), _full_spec((1, D)),
              _full_spec((1, D)), _full_spec((D, D)), _full_spec((1, D))],
    out_specs=_x_spec(),
    out_shape=jax.ShapeDtypeStruct((N, D), jnp.float32),
)


def kernel(x, edge_index, W1, b1, gamma, beta, W2, b2):
    src = edge_index[0].astype(jnp.int32)
    dst = edge_index[1].astype(jnp.int32)
    b1r = b1.reshape(1, D)
    gmr = gamma.reshape(1, D)
    btr = beta.reshape(1, D)
    b2r = b2.reshape(1, D)

    degp = _deg_sc(dst)
    g0 = _prep_tc(degp, x)
    p = _hop_sc(g0.reshape(2 * N, DH), src, dst)
    g1 = _comb_tc(degp, g0, p)
    p2 = _hop_sc(g1.reshape(2 * N, DH), src, dst)
    out = _final_tc(degp, g1, p2, W1, b1r, gmr, btr, W2, b2r)
    return out


# speedup vs baseline: 23.4484x; 2.6572x over previous
import functools

import jax
import jax.numpy as jnp
import numpy as np
from jax import lax
from jax.experimental import pallas as pl
from jax.experimental.pallas import tpu as pltpu
from jax.experimental.pallas import tpu_sc as plsc

N = 10000
NP = 10240
E = 320000
D = 128
DH = D // 2
BN_EPS = 1e-5

NC = 2
NS = 16
NW = NC * NS

K = 80
ZR = 128
SLAB = NP // NS

EPT_DEG = E // NW
CH_DEG = EPT_DEG // K
EPT_HOP = E // NS
CH_HOP = EPT_HOP // K

_mesh = plsc.VectorSubcoreMesh(core_axis_name="c", subcore_axis_name="s")
_sc_params = pltpu.CompilerParams(use_tc_tiling_on_sc=False)


@functools.partial(
    pl.kernel,
    out_type=jax.ShapeDtypeStruct((NC, NP, 16), jnp.float32),
    mesh=_mesh,
    scratch_types=[
        pltpu.VMEM((CH_DEG, K), jnp.int32),
        pltpu.VMEM((K, 16), jnp.float32),
        pltpu.VMEM((ZR, 16), jnp.float32),
        pltpu.VMEM_SHARED((NP, 16), jnp.float32),
    ],
    compiler_params=_sc_params,
)
def _deg_sc(dst_hbm, out_hbm, dst_v, ones_v, zbuf, acc):
    cid = lax.axis_index("c")
    sid = lax.axis_index("s")
    wid = sid * NC + cid

    def fill(r, carry):
        zbuf[r, :] = jnp.zeros((16,), jnp.float32)
        return carry

    lax.fori_loop(0, ZR, fill, 0)

    def fill1(r, carry):
        ones_v[r, :] = jnp.ones((16,), jnp.float32)
        return carry

    lax.fori_loop(0, K, fill1, 0)

    for z in range(SLAB // ZR):
        pltpu.sync_copy(zbuf, acc.at[pl.ds(sid * SLAB + z * ZR, ZR)])
    plsc.subcore_barrier()

    pltpu.sync_copy(dst_hbm.at[wid], dst_v)

    def body(c, carry):
        pltpu.sync_copy(ones_v, acc.at[dst_v.at[c]], add=True)
        return carry

    lax.fori_loop(0, CH_DEG, body, 0)

    plsc.subcore_barrier()
    pltpu.sync_copy(acc.at[pl.ds(sid * SLAB, SLAB)],
                    out_hbm.at[cid, pl.ds(sid * SLAB, SLAB)])


@functools.partial(
    pl.kernel,
    out_type=jax.ShapeDtypeStruct((NC, NP, DH), jnp.float32),
    mesh=_mesh,
    scratch_types=[
        pltpu.VMEM((CH_HOP, K), jnp.int32),
        pltpu.VMEM((CH_HOP, K), jnp.int32),
        pltpu.VMEM((2, K, DH), jnp.float32),
        pltpu.VMEM((ZR, DH), jnp.float32),
        pltpu.VMEM_SHARED((NP, DH), jnp.float32),
        pltpu.SemaphoreType.DMA,
        pltpu.SemaphoreType.DMA,
    ],
    compiler_params=_sc_params,
)
def _hop_sc(g_hbm, src_hbm, dst_hbm, out_hbm, src_v, dst_v, rows_v, zbuf, acc,
            sem0, sem1):
    cid = lax.axis_index("c")
    sid = lax.axis_index("s")

    def fill(r, carry):
        for j in range(DH // 16):
            zbuf[r, pl.ds(j * 16, 16)] = jnp.zeros((16,), jnp.float32)
        return carry

    lax.fori_loop(0, ZR, fill, 0)
    for z in range(SLAB // ZR):
        pltpu.sync_copy(zbuf, acc.at[pl.ds(sid * SLAB + z * ZR, ZR)])
    plsc.subcore_barrier()

    off = cid * N
    pltpu.sync_copy(src_hbm.at[sid], src_v)
    pltpu.sync_copy(dst_hbm.at[sid], dst_v)

    def adj(c, carry):
        for j in range(K // 16):
            sl = pl.ds(j * 16, 16)
            src_v[c, sl] = src_v[c, sl] + off
        return carry

    lax.fori_loop(0, CH_HOP, adj, 0)

    sems = (sem0, sem1)
    pltpu.async_copy(g_hbm.at[src_v.at[0]], rows_v.at[0], sem0)

    def body(h, carry):
        c0 = 2 * h
        for b in range(2):
            c = c0 + b
            nxt = c + 1

            @pl.when(nxt < CH_HOP)
            def _():
                pltpu.async_copy(g_hbm.at[src_v.at[nxt]], rows_v.at[1 - b],
                                 sems[1 - b])

            pltpu.make_async_copy(g_hbm.at[src_v.at[c]], rows_v.at[b],
                                  sems[b]).wait()
            pltpu.sync_copy(rows_v.at[b], acc.at[dst_v.at[c]], add=True)
        return carry

    lax.fori_loop(0, CH_HOP // 2, body, 0)

    plsc.subcore_barrier()
    pltpu.sync_copy(acc.at[pl.ds(sid * SLAB, SLAB)],
                    out_hbm.at[cid, pl.ds(sid * SLAB, SLAB)])


_BR = 1000


def _prep_body(dp_ref, x_ref, o_ref):
    deg = dp_ref[0, :, :1] + dp_ref[1, :, :1] + 1.0
    g = x_ref[...] * lax.rsqrt(deg)
    o_ref[0] = g[:, :DH]
    o_ref[1] = g[:, DH:]


def _comb_body(dp_ref, g_ref, p_ref, o_ref):
    deg = dp_ref[0, :, :1] + dp_ref[1, :, :1] + 1.0
    o_ref[0] = (p_ref[0] + g_ref[0]) / deg
    o_ref[1] = (p_ref[1] + g_ref[1]) / deg


def _final_body(dp_ref, g_ref, p_ref, w1_ref, b1_ref, gm_ref, bt_ref,
                w2_ref, b2_ref, o_ref):
    deg = dp_ref[0, :, :1] + dp_ref[1, :, :1] + 1.0
    h2 = jnp.concatenate([p_ref[0] + g_ref[0], p_ref[1] + g_ref[1]], axis=1)
    h2 = h2 * lax.rsqrt(deg)
    t = lax.dot_general(h2, w1_ref[...], (((1,), (1,)), ((), ())),
                        preferred_element_type=jnp.float32)
    bn_scale = np.float32(1.0 / np.sqrt(1.0 + BN_EPS))
    t = (t + b1_ref[...]) * (gm_ref[...] * bn_scale) + bt_ref[...]
    t = jnp.maximum(t, 0.0)
    o_ref[...] = lax.dot_general(t, w2_ref[...], (((1,), (1,)), ((), ())),
                                 preferred_element_type=jnp.float32) + b2_ref[...]


def _dp_spec():
    return pl.BlockSpec((NC, _BR, 16), lambda i: (0, i, 0))


def _x_spec():
    return pl.BlockSpec((_BR, D), lambda i: (i, 0))


def _g_spec():
    return pl.BlockSpec((2, _BR, DH), lambda i: (0, i, 0))


def _p_spec():
    return pl.BlockSpec((NC, _BR, DH), lambda i: (0, i, 0))


def _full_spec(shape):
    nd = len(shape)
    return pl.BlockSpec(shape, lambda i, _nd=nd: (0,) * _nd)


_prep_tc = pl.pallas_call(
    _prep_body,
    grid=(N // _BR,),
    in_specs=[_dp_spec(), _x_spec()],
    out_specs=_g_spec(),
    out_shape=jax.ShapeDtypeStruct((2, N, DH), jnp.float32),
)

_comb_tc = pl.pallas_call(
    _comb_body,
    grid=(N // _BR,),
    in_specs=[_dp_spec(), _g_spec(), _p_spec()],
    out_specs=_g_spec(),
    out_shape=jax.ShapeDtypeStruct((2, N, DH), jnp.float32),
)

_final_tc = pl.pallas_call(
    _final_body,
    grid=(N // _BR,),
    in_specs=[_dp_spec(), _g_spec(), _p_spec(),
              _full_spec((D, D)), _full_spec((1, D)), _full_spec((1, D)),
              _full_spec((1, D)), _full_spec((D, D)), _full_spec((1, D))],
    out_specs=_x_spec(),
    out_shape=jax.ShapeDtypeStruct((N, D), jnp.float32),
)


def kernel(x, edge_index, W1, b1, gamma, beta, W2, b2):
    src = edge_index[0].astype(jnp.int32)
    dst = edge_index[1].astype(jnp.int32)
    b1r = b1.reshape(1, D)
    gmr = gamma.reshape(1, D)
    btr = beta.reshape(1, D)
    b2r = b2.reshape(1, D)

    src3 = src.reshape(NS, CH_HOP, K)
    dst3 = dst.reshape(NS, CH_HOP, K)
    dst3d = dst.reshape(NW, CH_DEG, K)

    degp = _deg_sc(dst3d)
    g0 = _prep_tc(degp, x)
    p = _hop_sc(g0.reshape(2 * N, DH), src3, dst3)
    g1 = _comb_tc(degp, g0, p)
    p2 = _hop_sc(g1.reshape(2 * N, DH), src3, dst3)
    out = _final_tc(degp, g1, p2, W1, b1r, gmr, btr, W2, b2r)
    return out


# speedup vs baseline: 31.8634x; 1.3589x over previous
import functools

import jax
import jax.numpy as jnp
import numpy as np
from jax import lax
from jax.experimental import pallas as pl
from jax.experimental.pallas import tpu as pltpu
from jax.experimental.pallas import tpu_sc as plsc

N = 10000
NP = 10240
E = 320000
D = 128
DH = D // 2
BN_EPS = 1e-5

NC = 2
NS = 16
NW = NC * NS

K = 80
ZR = 128
SLAB = NP // NS

EPT_DEG = E // NW
CH_DEG = EPT_DEG // K
EPT_HOP = E // NS
CH_HOP = EPT_HOP // K
NBUF = 5

_mesh = plsc.VectorSubcoreMesh(core_axis_name="c", subcore_axis_name="s")
_sc_params = pltpu.CompilerParams(use_tc_tiling_on_sc=False)


@functools.partial(
    pl.kernel,
    out_type=jax.ShapeDtypeStruct((NC, NP, 16), jnp.float32),
    mesh=_mesh,
    scratch_types=[
        pltpu.VMEM((CH_DEG, K), jnp.int32),
        pltpu.VMEM((K, 16), jnp.float32),
        pltpu.VMEM((ZR, 16), jnp.float32),
        pltpu.VMEM_SHARED((NP, 16), jnp.float32),
    ],
    compiler_params=_sc_params,
)
def _deg_sc(dst_hbm, out_hbm, dst_v, ones_v, zbuf, acc):
    cid = lax.axis_index("c")
    sid = lax.axis_index("s")
    wid = sid * NC + cid

    def fill(r, carry):
        zbuf[r, :] = jnp.zeros((16,), jnp.float32)
        return carry

    lax.fori_loop(0, ZR, fill, 0)

    def fill1(r, carry):
        ones_v[r, :] = jnp.ones((16,), jnp.float32)
        return carry

    lax.fori_loop(0, K, fill1, 0)

    for z in range(SLAB // ZR):
        pltpu.sync_copy(zbuf, acc.at[pl.ds(sid * SLAB + z * ZR, ZR)])
    plsc.subcore_barrier()

    pltpu.sync_copy(dst_hbm.at[wid], dst_v)

    def body(c, carry):
        pltpu.sync_copy(ones_v, acc.at[dst_v.at[c]], add=True)
        return carry

    lax.fori_loop(0, CH_DEG, body, 0)

    plsc.subcore_barrier()
    pltpu.sync_copy(acc.at[pl.ds(sid * SLAB, SLAB)],
                    out_hbm.at[cid, pl.ds(sid * SLAB, SLAB)])


@functools.partial(
    pl.kernel,
    out_type=jax.ShapeDtypeStruct((NC, NP, DH), jnp.float32),
    mesh=_mesh,
    scratch_types=[
        pltpu.VMEM((CH_HOP, K), jnp.int32),
        pltpu.VMEM((CH_HOP, K), jnp.int32),
        pltpu.VMEM((NBUF, K, DH), jnp.float32),
        pltpu.VMEM((ZR, DH), jnp.float32),
        pltpu.VMEM_SHARED((NP, DH), jnp.float32),
        [pltpu.SemaphoreType.DMA] * NBUF,
        [pltpu.SemaphoreType.DMA] * NBUF,
    ],
    compiler_params=_sc_params,
)
def _hop_sc(g_hbm, src_hbm, dst_hbm, out_hbm, src_v, dst_v, rows_v, zbuf, acc,
            gsems, ssems):
    cid = lax.axis_index("c")
    sid = lax.axis_index("s")

    def fill(r, carry):
        for j in range(DH // 16):
            zbuf[r, pl.ds(j * 16, 16)] = jnp.zeros((16,), jnp.float32)
        return carry

    lax.fori_loop(0, ZR, fill, 0)
    for z in range(SLAB // ZR):
        pltpu.sync_copy(zbuf, acc.at[pl.ds(sid * SLAB + z * ZR, ZR)])
    plsc.subcore_barrier()

    off = cid * N
    pltpu.sync_copy(src_hbm.at[sid], src_v)
    pltpu.sync_copy(dst_hbm.at[sid], dst_v)

    def adj(c, carry):
        for j in range(K // 16):
            sl = pl.ds(j * 16, 16)
            src_v[c, sl] = src_v[c, sl] + off
        return carry

    lax.fori_loop(0, CH_HOP, adj, 0)

    def gather_start(c, b):
        pltpu.async_copy(g_hbm.at[src_v.at[c]], rows_v.at[b], gsems[b])

    def gather_wait(c, b):
        pltpu.make_async_copy(g_hbm.at[src_v.at[c]], rows_v.at[b],
                              gsems[b]).wait()

    def scat_start(c, b):
        pltpu.async_copy(rows_v.at[b], acc.at[dst_v.at[c]], ssems[b], add=True)

    def scat_wait(c, b):
        pltpu.make_async_copy(rows_v.at[b], acc.at[dst_v.at[c]],
                              ssems[b]).wait()

    for b in range(NBUF - 1):
        gather_start(b, b)

    def body(h, carry):
        c0 = h * NBUF
        for b in range(NBUF):
            c = c0 + b
            gather_wait(c, b)
            scat_start(c, b)
            bp = (b + NBUF - 1) % NBUF
            cp = c + NBUF - 1

            @pl.when(cp < CH_HOP)
            def _():
                @pl.when(c > 0)
                def _():
                    scat_wait(c - 1, bp)

                gather_start(cp, bp)
        return carry

    lax.fori_loop(0, CH_HOP // NBUF, body, 0)

    for b in range(NBUF):
        c = CH_HOP - NBUF + b
        scat_wait(c, b % NBUF)

    plsc.subcore_barrier()
    pltpu.sync_copy(acc.at[pl.ds(sid * SLAB, SLAB)],
                    out_hbm.at[cid, pl.ds(sid * SLAB, SLAB)])


_BR = 1000


def _prep_body(dp_ref, x_ref, o_ref):
    deg = dp_ref[0, :, :1] + dp_ref[1, :, :1] + 1.0
    g = x_ref[...] * lax.rsqrt(deg)
    o_ref[0] = g[:, :DH]
    o_ref[1] = g[:, DH:]


def _comb_body(dp_ref, g_ref, p_ref, o_ref):
    deg = dp_ref[0, :, :1] + dp_ref[1, :, :1] + 1.0
    o_ref[0] = (p_ref[0] + g_ref[0]) / deg
    o_ref[1] = (p_ref[1] + g_ref[1]) / deg


def _final_body(dp_ref, g_ref, p_ref, w1_ref, b1_ref, gm_ref, bt_ref,
                w2_ref, b2_ref, o_ref):
    deg = dp_ref[0, :, :1] + dp_ref[1, :, :1] + 1.0
    h2 = jnp.concatenate([p_ref[0] + g_ref[0], p_ref[1] + g_ref[1]], axis=1)
    h2 = h2 * lax.rsqrt(deg)
    t = lax.dot_general(h2, w1_ref[...], (((1,), (1,)), ((), ())),
                        preferred_element_type=jnp.float32)
    bn_scale = np.float32(1.0 / np.sqrt(1.0 + BN_EPS))
    t = (t + b1_ref[...]) * (gm_ref[...] * bn_scale) + bt_ref[...]
    t = jnp.maximum(t, 0.0)
    o_ref[...] = lax.dot_general(t, w2_ref[...], (((1,), (1,)), ((), ())),
                                 preferred_element_type=jnp.float32) + b2_ref[...]


def _dp_spec():
    return pl.BlockSpec((NC, _BR, 16), lambda i: (0, i, 0))


def _x_spec():
    return pl.BlockSpec((_BR, D), lambda i: (i, 0))


def _g_spec():
    return pl.BlockSpec((2, _BR, DH), lambda i: (0, i, 0))


def _p_spec():
    return pl.BlockSpec((NC, _BR, DH), lambda i: (0, i, 0))


def _full_spec(shape):
    nd = len(shape)
    return pl.BlockSpec(shape, lambda i, _nd=nd: (0,) * _nd)


_prep_tc = pl.pallas_call(
    _prep_body,
    grid=(N // _BR,),
    in_specs=[_dp_spec(), _x_spec()],
    out_specs=_g_spec(),
    out_shape=jax.ShapeDtypeStruct((2, N, DH), jnp.float32),
)

_comb_tc = pl.pallas_call(
    _comb_body,
    grid=(N // _BR,),
    in_specs=[_dp_spec(), _g_spec(), _p_spec()],
    out_specs=_g_spec(),
    out_shape=jax.ShapeDtypeStruct((2, N, DH), jnp.float32),
)

_final_tc = pl.pallas_call(
    _final_body,
    grid=(N // _BR,),
    in_specs=[_dp_spec(), _g_spec(), _p_spec(),
              _full_spec((D, D)), _full_spec((1, D)), _full_spec((1, D)),
              _full_spec((1, D)), _full_spec((D, D)), _full_spec((1, D))],
    out_specs=_x_spec(),
    out_shape=jax.ShapeDtypeStruct((N, D), jnp.float32),
)


def kernel(x, edge_index, W1, b1, gamma, beta, W2, b2):
    src = edge_index[0].astype(jnp.int32)
    dst = edge_index[1].astype(jnp.int32)
    b1r = b1.reshape(1, D)
    gmr = gamma.reshape(1, D)
    btr = beta.reshape(1, D)
    b2r = b2.reshape(1, D)

    src3 = src.reshape(NS, CH_HOP, K)
    dst3 = dst.reshape(NS, CH_HOP, K)
    dst3d = dst.reshape(NW, CH_DEG, K)

    degp = _deg_sc(dst3d)
    g0 = _prep_tc(degp, x)
    p = _hop_sc(g0.reshape(2 * N, DH), src3, dst3)
    g1 = _comb_tc(degp, g0, p)
    p2 = _hop_sc(g1.reshape(2 * N, DH), src3, dst3)
    out = _final_tc(degp, g1, p2, W1, b1r, gmr, btr, W2, b2r)
    return out


# speedup vs baseline: 32.4603x; 1.0187x over previous
import functools

import jax
import jax.numpy as jnp
import numpy as np
from jax import lax
from jax.experimental import pallas as pl
from jax.experimental.pallas import tpu as pltpu
from jax.experimental.pallas import tpu_sc as plsc

N = 10000
NP = 10240
E = 320000
D = 128
DH = D // 2
BN_EPS = 1e-5

NC = 2
NS = 16
NW = NC * NS

K = 80
ZR = 128
SLAB = NP // NS

EPT_DEG = E // NW
CH_DEG = EPT_DEG // K
EPT_HOP = E // NS
CH_HOP = EPT_HOP // K
NBUF = 5

_mesh = plsc.VectorSubcoreMesh(core_axis_name="c", subcore_axis_name="s")
_sc_params = pltpu.CompilerParams(use_tc_tiling_on_sc=False)


@functools.partial(
    pl.kernel,
    out_type=jax.ShapeDtypeStruct((NC, NP, 16), jnp.float32),
    mesh=_mesh,
    scratch_types=[
        pltpu.VMEM((CH_HOP, K), jnp.int32),
        pltpu.VMEM((K, 16), jnp.float32),
        pltpu.VMEM((ZR, 16), jnp.float32),
        pltpu.VMEM_SHARED((NP, 16), jnp.float32),
        [pltpu.SemaphoreType.DMA] * NBUF,
    ],
    compiler_params=_sc_params,
)
def _deg_sc(ei_hbm, out_hbm, dst_v, ones_v, zbuf, acc, ssems):
    cid = lax.axis_index("c")
    sid = lax.axis_index("s")

    def fill(r, carry):
        zbuf[r, :] = jnp.zeros((16,), jnp.float32)
        return carry

    lax.fori_loop(0, ZR, fill, 0)

    def fill1(r, carry):
        ones_v[r, :] = jnp.ones((16,), jnp.float32)
        return carry

    lax.fori_loop(0, K, fill1, 0)

    for z in range(SLAB // ZR):
        pltpu.sync_copy(zbuf, acc.at[pl.ds(sid * SLAB + z * ZR, ZR)])
    plsc.subcore_barrier()

    pltpu.sync_copy(ei_hbm.at[1, sid], dst_v)

    def scat_start(c, b):
        pltpu.async_copy(ones_v, acc.at[dst_v.at[c]], ssems[b], add=True)

    def scat_wait(c, b):
        pltpu.make_async_copy(ones_v, acc.at[dst_v.at[c]], ssems[b]).wait()

    def body(h, carry):
        c0 = h * NBUF
        for b in range(NBUF):
            c = c0 + b

            @pl.when(c >= NBUF)
            def _():
                scat_wait(c - NBUF, b)

            scat_start(c, b)
        return carry

    lax.fori_loop(0, CH_HOP // NBUF, body, 0)
    for b in range(NBUF):
        scat_wait(CH_HOP - NBUF + b, b)

    plsc.subcore_barrier()
    pltpu.sync_copy(acc.at[pl.ds(sid * SLAB, SLAB)],
                    out_hbm.at[cid, pl.ds(sid * SLAB, SLAB)])


@functools.partial(
    pl.kernel,
    out_type=jax.ShapeDtypeStruct((NC, NP, DH), jnp.float32),
    mesh=_mesh,
    scratch_types=[
        pltpu.VMEM((CH_HOP, K), jnp.int32),
        pltpu.VMEM((CH_HOP, K), jnp.int32),
        pltpu.VMEM((NBUF, K, DH), jnp.float32),
        pltpu.VMEM((ZR, DH), jnp.float32),
        pltpu.VMEM_SHARED((NP, DH), jnp.float32),
        [pltpu.SemaphoreType.DMA] * NBUF,
        [pltpu.SemaphoreType.DMA] * NBUF,
    ],
    compiler_params=_sc_params,
)
def _hop_sc(g_hbm, ei_hbm, out_hbm, src_v, dst_v, rows_v, zbuf, acc,
            gsems, ssems):
    cid = lax.axis_index("c")
    sid = lax.axis_index("s")

    def fill(r, carry):
        for j in range(DH // 16):
            zbuf[r, pl.ds(j * 16, 16)] = jnp.zeros((16,), jnp.float32)
        return carry

    lax.fori_loop(0, ZR, fill, 0)
    for z in range(SLAB // ZR):
        pltpu.sync_copy(zbuf, acc.at[pl.ds(sid * SLAB + z * ZR, ZR)])
    plsc.subcore_barrier()

    off = cid * N
    pltpu.sync_copy(ei_hbm.at[0, sid], src_v)
    pltpu.sync_copy(ei_hbm.at[1, sid], dst_v)

    def adj(c, carry):
        for j in range(K // 16):
            sl = pl.ds(j * 16, 16)
            src_v[c, sl] = src_v[c, sl] + off
        return carry

    lax.fori_loop(0, CH_HOP, adj, 0)

    def gather_start(c, b):
        pltpu.async_copy(g_hbm.at[src_v.at[c]], rows_v.at[b], gsems[b])

    def gather_wait(c, b):
        pltpu.make_async_copy(g_hbm.at[src_v.at[c]], rows_v.at[b],
                              gsems[b]).wait()

    def scat_start(c, b):
        pltpu.async_copy(rows_v.at[b], acc.at[dst_v.at[c]], ssems[b], add=True)

    def scat_wait(c, b):
        pltpu.make_async_copy(rows_v.at[b], acc.at[dst_v.at[c]],
                              ssems[b]).wait()

    for b in range(NBUF - 1):
        gather_start(b, b)

    def body(h, carry):
        c0 = h * NBUF
        for b in range(NBUF):
            c = c0 + b
            gather_wait(c, b)
            scat_start(c, b)
            bp = (b + NBUF - 1) % NBUF
            cp = c + NBUF - 1

            @pl.when(cp < CH_HOP)
            def _():
                @pl.when(c > 0)
                def _():
                    scat_wait(c - 1, bp)

                gather_start(cp, bp)
        return carry

    lax.fori_loop(0, CH_HOP // NBUF, body, 0)

    for b in range(NBUF):
        c = CH_HOP - NBUF + b
        scat_wait(c, b % NBUF)

    plsc.subcore_barrier()
    pltpu.sync_copy(acc.at[pl.ds(sid * SLAB, SLAB)],
                    out_hbm.at[cid, pl.ds(sid * SLAB, SLAB)])


_BR = 2000
_NB = N // _BR


def _prep_body(dp_ref, x_ref, o_ref):
    h = pl.program_id(0)
    deg = dp_ref[0, :, :1] + 1.0
    g = x_ref[...] * lax.rsqrt(deg)

    @pl.when(h == 0)
    def _():
        o_ref[...] = g[:, :DH]

    @pl.when(h == 1)
    def _():
        o_ref[...] = g[:, DH:]


def _comb_body(dp_ref, g_ref, p_ref, o_ref):
    deg = dp_ref[0, :, :1] + 1.0
    o_ref[...] = (p_ref[0] + g_ref[...]) / deg


def _final_body(dp_ref, glo_ref, ghi_ref, plo_ref, phi_ref,
                w1_ref, b1_ref, gm_ref, bt_ref, w2_ref, b2_ref, o_ref):
    deg = dp_ref[0, :, :1] + 1.0
    h2 = jnp.concatenate([plo_ref[0] + glo_ref[...],
                          phi_ref[0] + ghi_ref[...]], axis=1)
    h2 = h2 * lax.rsqrt(deg)
    t = lax.dot_general(h2, w1_ref[...], (((1,), (1,)), ((), ())),
                        preferred_element_type=jnp.float32)
    bn_scale = np.float32(1.0 / np.sqrt(1.0 + BN_EPS))
    t = (t + b1_ref[...]) * (gm_ref[...] * bn_scale) + bt_ref[...]
    t = jnp.maximum(t, 0.0)
    o_ref[...] = lax.dot_general(t, w2_ref[...], (((1,), (1,)), ((), ())),
                                 preferred_element_type=jnp.float32) + b2_ref[...]


def _full_spec(shape, ng):
    nd = len(shape)
    if ng == 1:
        return pl.BlockSpec(shape, lambda i, _nd=nd: (0,) * _nd)
    return pl.BlockSpec(shape, lambda h, i, _nd=nd: (0,) * _nd)


_prep_tc = pl.pallas_call(
    _prep_body,
    grid=(2, _NB),
    in_specs=[pl.BlockSpec((NC, _BR, 16), lambda h, i: (0, i, 0)),
              pl.BlockSpec((_BR, D), lambda h, i: (i, 0))],
    out_specs=pl.BlockSpec((_BR, DH), lambda h, i: (h * _NB + i, 0)),
    out_shape=jax.ShapeDtypeStruct((2 * N, DH), jnp.float32),
)

_comb_tc = pl.pallas_call(
    _comb_body,
    grid=(2, _NB),
    in_specs=[pl.BlockSpec((NC, _BR, 16), lambda h, i: (0, i, 0)),
              pl.BlockSpec((_BR, DH), lambda h, i: (h * _NB + i, 0)),
              pl.BlockSpec((1, _BR, DH), lambda h, i: (h, i, 0))],
    out_specs=pl.BlockSpec((_BR, DH), lambda h, i: (h * _NB + i, 0)),
    out_shape=jax.ShapeDtypeStruct((2 * N, DH), jnp.float32),
)

_final_tc = pl.pallas_call(
    _final_body,
    grid=(_NB,),
    in_specs=[pl.BlockSpec((NC, _BR, 16), lambda i: (0, i, 0)),
              pl.BlockSpec((_BR, DH), lambda i: (i, 0)),
              pl.BlockSpec((_BR, DH), lambda i: (_NB + i, 0)),
              pl.BlockSpec((1, _BR, DH), lambda i: (0, i, 0)),
              pl.BlockSpec((1, _BR, DH), lambda i: (1, i, 0)),
              _full_spec((D, D), 1), _full_spec((1, D), 1),
              _full_spec((1, D), 1), _full_spec((1, D), 1),
              _full_spec((D, D), 1), _full_spec((1, D), 1)],
    out_specs=pl.BlockSpec((_BR, D), lambda i: (i, 0)),
    out_shape=jax.ShapeDtypeStruct((N, D), jnp.float32),
)


def kernel(x, edge_index, W1, b1, gamma, beta, W2, b2):
    ei_h = edge_index.astype(jnp.int32).reshape(2, NS, CH_HOP, K)
    b1r = b1.reshape(1, D)
    gmr = gamma.reshape(1, D)
    btr = beta.reshape(1, D)
    b2r = b2.reshape(1, D)

    degp = _deg_sc(ei_h)
    g0 = _prep_tc(degp, x)
    p = _hop_sc(g0, ei_h)
    g1 = _comb_tc(degp, g0, p)
    p2 = _hop_sc(g1, ei_h)
    out = _final_tc(degp, g1, g1, p2, p2, W1, b1r, gmr, btr, W2, b2r)
    return out


# speedup vs baseline: 32.6912x; 1.0071x over previous
import functools

import jax
import jax.numpy as jnp
import numpy as np
from jax import lax
from jax.experimental import pallas as pl
from jax.experimental.pallas import tpu as pltpu
from jax.experimental.pallas import tpu_sc as plsc

N = 10000
NP = 10240
E = 320000
D = 128
DH = D // 2
BN_EPS = 1e-5

NC = 2
NS = 16

K = 80
CH = (E // NS) // K
NBUF = 5
ZR = 64
SLAB = NP // NS
RSL = N // NS
SEG = 125

_mesh = plsc.VectorSubcoreMesh(core_axis_name="c", subcore_axis_name="s")
_sc_params = pltpu.CompilerParams(use_tc_tiling_on_sc=False)


def _rsqrt16(d):
    y = jnp.full((16,), 2.0 ** -10, jnp.float32)
    for k in range(9, 0, -1):
        y = jnp.where(d < 4.0 ** k, jnp.float32(2.0 ** -k), y)
    half = d * 0.5
    for _ in range(6):
        y = y * (1.5 - half * y * y)
    return y


@functools.partial(
    pl.kernel,
    out_type=(jax.ShapeDtypeStruct((NC, NP, 16), jnp.float32),
              jax.ShapeDtypeStruct((2 * N, DH), jnp.float32)),
    mesh=_mesh,
    scratch_types=[
        pltpu.VMEM((CH, K), jnp.int32),
        pltpu.VMEM((K, 16), jnp.float32),
        pltpu.VMEM((ZR, 16), jnp.float32),
        pltpu.VMEM((SEG, 16), jnp.float32),
        pltpu.VMEM((SEG, DH), jnp.float32),
        pltpu.VMEM_SHARED((NP, 16), jnp.float32),
        [pltpu.SemaphoreType.DMA] * NBUF,
    ],
    compiler_params=_sc_params,
)
def _deg_sc(ei_hbm, x_hbm, dp_hbm, g0_hbm, dst_v, ones_v, zbuf, degb, xb,
            acc, ssems):
    cid = lax.axis_index("c")
    sid = lax.axis_index("s")

    def fill(r, carry):
        zbuf[r, :] = jnp.zeros((16,), jnp.float32)
        return carry

    lax.fori_loop(0, ZR, fill, 0)

    def fill1(r, carry):
        ones_v[r, :] = jnp.ones((16,), jnp.float32)
        return carry

    lax.fori_loop(0, K, fill1, 0)

    for z in range(SLAB // ZR):
        pltpu.sync_copy(zbuf, acc.at[pl.ds(sid * SLAB + z * ZR, ZR)])
    plsc.subcore_barrier()

    pltpu.sync_copy(ei_hbm.at[1, sid], dst_v)

    def scat_start(c, b):
        pltpu.async_copy(ones_v, acc.at[dst_v.at[c]], ssems[b], add=True)

    def scat_wait(c, b):
        pltpu.make_async_copy(ones_v, acc.at[dst_v.at[c]], ssems[b]).wait()

    def body(h, carry):
        c0 = h * NBUF
        for b in range(NBUF):
            c = c0 + b

            @pl.when(c >= NBUF)
            def _():
                scat_wait(c - NBUF, b)

            scat_start(c, b)
        return carry

    lax.fori_loop(0, CH // NBUF, body, 0)
    for b in range(NBUF):
        scat_wait(CH - NBUF + b, b)

    plsc.subcore_barrier()
    pltpu.sync_copy(acc.at[pl.ds(sid * SLAB, SLAB)],
                    dp_hbm.at[cid, pl.ds(sid * SLAB, SLAB)])

    for z in range(RSL // SEG):
        r0 = sid * RSL + z * SEG
        pltpu.sync_copy(acc.at[pl.ds(r0, SEG)], degb)
        pltpu.sync_copy(x_hbm.at[pl.ds(r0, SEG), pl.ds(cid * DH, DH)], xb)

        def scale(r, carry):
            dis = _rsqrt16(degb[r, :] + 1.0)
            for j in range(DH // 16):
                sl = pl.ds(j * 16, 16)
                xb[r, sl] = xb[r, sl] * dis
            return carry

        lax.fori_loop(0, SEG, scale, 0)
        pltpu.sync_copy(xb, g0_hbm.at[pl.ds(cid * N + r0, SEG)])


def _hop_ring(g_hbm, src_v, dst_v, rows_v, acc, gsems, ssems):
    def gather_start(c, b):
        pltpu.async_copy(g_hbm.at[src_v.at[c]], rows_v.at[b], gsems[b])

    def gather_wait(c, b):
        pltpu.make_async_copy(g_hbm.at[src_v.at[c]], rows_v.at[b],
                              gsems[b]).wait()

    def scat_start(c, b):
        pltpu.async_copy(rows_v.at[b], acc.at[dst_v.at[c]], ssems[b], add=True)

    def scat_wait(c, b):
        pltpu.make_async_copy(rows_v.at[b], acc.at[dst_v.at[c]],
                              ssems[b]).wait()

    for b in range(NBUF - 1):
        gather_start(b, b)

    def body(h, carry):
        c0 = h * NBUF
        for b in range(NBUF):
            c = c0 + b
            gather_wait(c, b)
            scat_start(c, b)
            bp = (b + NBUF - 1) % NBUF
            cp = c + NBUF - 1

            @pl.when(cp < CH)
            def _():
                @pl.when(c > 0)
                def _():
                    scat_wait(c - 1, bp)

                gather_start(cp, bp)
        return carry

    lax.fori_loop(0, CH // NBUF, body, 0)
    for b in range(NBUF):
        scat_wait(CH - NBUF + b, b)


def _fill_zbuf(zbuf):
    def fill(r, carry):
        for j in range(DH // 16):
            zbuf[r, pl.ds(j * 16, 16)] = jnp.zeros((16,), jnp.float32)
        return carry

    lax.fori_loop(0, ZR, fill, 0)


def _load_adjusted_idx(ei_hbm, src_v, dst_v, cid, sid):
    off = cid * N
    pltpu.sync_copy(ei_hbm.at[0, sid], src_v)
    pltpu.sync_copy(ei_hbm.at[1, sid], dst_v)

    def adj(c, carry):
        for j in range(K // 16):
            sl = pl.ds(j * 16, 16)
            src_v[c, sl] = src_v[c, sl] + off
        return carry

    lax.fori_loop(0, CH, adj, 0)


@functools.partial(
    pl.kernel,
    out_type=jax.ShapeDtypeStruct((NC, NP, DH), jnp.float32),
    mesh=_mesh,
    scratch_types=[
        pltpu.VMEM((CH, K), jnp.int32),
        pltpu.VMEM((CH, K), jnp.int32),
        pltpu.VMEM((NBUF, K, DH), jnp.float32),
        pltpu.VMEM((ZR, DH), jnp.float32),
        pltpu.VMEM_SHARED((NP, DH), jnp.float32),
        [pltpu.SemaphoreType.DMA] * NBUF,
        [pltpu.SemaphoreType.DMA] * NBUF,
    ],
    compiler_params=_sc_params,
)
def _hop_sc(g_hbm, ei_hbm, out_hbm, src_v, dst_v, rows_v, zbuf, acc,
            gsems, ssems):
    cid = lax.axis_index("c")
    sid = lax.axis_index("s")

    _fill_zbuf(zbuf)
    for z in range(SLAB // ZR):
        pltpu.sync_copy(zbuf, acc.at[pl.ds(sid * SLAB + z * ZR, ZR)])
    _load_adjusted_idx(ei_hbm, src_v, dst_v, cid, sid)
    plsc.subcore_barrier()

    _hop_ring(g_hbm, src_v, dst_v, rows_v, acc, gsems, ssems)

    plsc.subcore_barrier()
    pltpu.sync_copy(acc.at[pl.ds(sid * SLAB, SLAB)],
                    out_hbm.at[cid, pl.ds(sid * SLAB, SLAB)])


@functools.partial(
    pl.kernel,
    out_type=(jax.ShapeDtypeStruct((2 * N, DH), jnp.float32),
              jax.ShapeDtypeStruct((2 * N, DH), jnp.float32)),
    mesh=_mesh,
    scratch_types=[
        pltpu.VMEM((CH, K), jnp.int32),
        pltpu.VMEM((CH, K), jnp.int32),
        pltpu.VMEM((NBUF, K, DH), jnp.float32),
        pltpu.VMEM((ZR, DH), jnp.float32),
        pltpu.VMEM((SEG, 16), jnp.float32),
        pltpu.VMEM((SEG, DH), jnp.float32),
        pltpu.VMEM((SEG, DH), jnp.float32),
        pltpu.VMEM_SHARED((NP, DH), jnp.float32),
        [pltpu.SemaphoreType.DMA] * NBUF,
        [pltpu.SemaphoreType.DMA] * NBUF,
    ],
    compiler_params=_sc_params,
)
def _combhop_sc(dp_hbm, g0_hbm, p_hbm, ei_hbm, g1_hbm, h2_hbm, src_v, dst_v,
                rows_v, zbuf, degb, pb, g1b, acc, gsems, ssems):
    cid = lax.axis_index("c")
    sid = lax.axis_index("s")

    _fill_zbuf(zbuf)
    for z in range(SLAB // ZR):
        pltpu.sync_copy(zbuf, acc.at[pl.ds(sid * SLAB + z * ZR, ZR)])
    _load_adjusted_idx(ei_hbm, src_v, dst_v, cid, sid)

    for z in range(RSL // SEG):
        r0 = sid * RSL + z * SEG
        pltpu.sync_copy(dp_hbm.at[cid, pl.ds(r0, SEG)], degb)
        pltpu.sync_copy(p_hbm.at[cid, pl.ds(r0, SEG)], pb)
        pltpu.sync_copy(g0_hbm.at[pl.ds(cid * N + r0, SEG)], g1b)

        def comb(r, carry):
            rdeg = 1.0 / (degb[r, :] + 1.0)
            for j in range(DH // 16):
                sl = pl.ds(j * 16, 16)
                g1b[r, sl] = (pb[r, sl] + g1b[r, sl]) * rdeg
            return carry

        lax.fori_loop(0, SEG, comb, 0)
        pltpu.sync_copy(g1b, g1_hbm.at[pl.ds(cid * N + r0, SEG)])
    plsc.subcore_barrier()

    _hop_ring(g1_hbm, src_v, dst_v, rows_v, acc, gsems, ssems)
    plsc.subcore_barrier()

    for z in range(RSL // SEG):
        r0 = sid * RSL + z * SEG
        pltpu.sync_copy(acc.at[pl.ds(r0, SEG)], pb)
        pltpu.sync_copy(g1_hbm.at[pl.ds(cid * N + r0, SEG)], g1b)

        def fin(r, carry):
            for j in range(DH // 16):
                sl = pl.ds(j * 16, 16)
                pb[r, sl] = pb[r, sl] + g1b[r, sl]
            return carry

        lax.fori_loop(0, SEG, fin, 0)
        pltpu.sync_copy(pb, h2_hbm.at[pl.ds(cid * N + r0, SEG)])


_BR = 2000
_NB = N // _BR


def _final_body(dp_ref, glo_ref, ghi_ref, w1_ref, b1_ref, gm_ref, bt_ref,
                w2_ref, b2_ref, o_ref):
    deg = dp_ref[0, :, :1] + 1.0
    h2 = jnp.concatenate([glo_ref[...], ghi_ref[...]], axis=1)
    h2 = h2 * lax.rsqrt(deg)
    t = lax.dot_general(h2, w1_ref[...], (((1,), (1,)), ((), ())),
                        preferred_element_type=jnp.float32)
    bn_scale = np.float32(1.0 / np.sqrt(1.0 + BN_EPS))
    t = (t + b1_ref[...]) * (gm_ref[...] * bn_scale) + bt_ref[...]
    t = jnp.maximum(t, 0.0)
    o_ref[...] = lax.dot_general(t, w2_ref[...], (((1,), (1,)), ((), ())),
                                 preferred_element_type=jnp.float32) + b2_ref[...]


def _full_spec(shape):
    nd = len(shape)
    return pl.BlockSpec(shape, lambda i, _nd=nd: (0,) * _nd)


_final_tc = pl.pallas_call(
    _final_body,
    grid=(_NB,),
    in_specs=[pl.BlockSpec((NC, _BR, 16), lambda i: (0, i, 0)),
              pl.BlockSpec((_BR, DH), lambda i: (i, 0)),
              pl.BlockSpec((_BR, DH), lambda i: (_NB + i, 0)),
              _full_spec((D, D)), _full_spec((1, D)), _full_spec((1, D)),
              _full_spec((1, D)), _full_spec((D, D)), _full_spec((1, D))],
    out_specs=pl.BlockSpec((_BR, D), lambda i: (i, 0)),
    out_shape=jax.ShapeDtypeStruct((N, D), jnp.float32),
)


def kernel(x, edge_index, W1, b1, gamma, beta, W2, b2):
    ei = edge_index.astype(jnp.int32).reshape(2, NS, CH, K)
    b1r = b1.reshape(1, D)
    gmr = gamma.reshape(1, D)
    btr = beta.reshape(1, D)
    b2r = b2.reshape(1, D)
    dp, g0 = _deg_sc(ei, x)
    p = _hop_sc(g0, ei)
    g1, h2u = _combhop_sc(dp, g0, p, ei)
    del g1
    out = _final_tc(dp, h2u, h2u, W1, b1r, gmr, btr, W2, b2r)
    return out


# speedup vs baseline: 33.0175x; 1.0100x over previous
import functools

import jax
import jax.numpy as jnp
import numpy as np
from jax import lax
from jax.experimental import pallas as pl
from jax.experimental.pallas import tpu as pltpu
from jax.experimental.pallas import tpu_sc as plsc

N = 10000
NP = 10240
E = 320000
D = 128
DH = D // 2
BN_EPS = 1e-5

NC = 2
NS = 16

K = 80
CH = (E // NS) // K
NBUF = 5
ZR = 64
SLAB = NP // NS
RSL = N // NS
SEG = 125

_mesh = plsc.VectorSubcoreMesh(core_axis_name="c", subcore_axis_name="s")
_sc_params = pltpu.CompilerParams(use_tc_tiling_on_sc=False)


@functools.partial(
    pl.kernel,
    out_type=jax.ShapeDtypeStruct((NC, NP, 16), jnp.float32),
    mesh=_mesh,
    scratch_types=[
        pltpu.VMEM((CH, K), jnp.int32),
        pltpu.VMEM((K, 16), jnp.float32),
        pltpu.VMEM((ZR, 16), jnp.float32),
        pltpu.VMEM_SHARED((NP, 16), jnp.float32),
        [pltpu.SemaphoreType.DMA] * NBUF,
    ],
    compiler_params=_sc_params,
)
def _deg_sc(ei_hbm, dp_hbm, dst_v, ones_v, zbuf, acc, ssems):
    cid = lax.axis_index("c")
    sid = lax.axis_index("s")

    def fill(r, carry):
        zbuf[r, :] = jnp.zeros((16,), jnp.float32)
        return carry

    lax.fori_loop(0, ZR, fill, 0)

    def fill1(r, carry):
        ones_v[r, :] = jnp.ones((16,), jnp.float32)
        return carry

    lax.fori_loop(0, K, fill1, 0)

    for z in range(SLAB // ZR):
        pltpu.sync_copy(zbuf, acc.at[pl.ds(sid * SLAB + z * ZR, ZR)])
    plsc.subcore_barrier()

    pltpu.sync_copy(ei_hbm.at[1, sid], dst_v)

    def scat_start(c, b):
        pltpu.async_copy(ones_v, acc.at[dst_v.at[c]], ssems[b], add=True)

    def scat_wait(c, b):
        pltpu.make_async_copy(ones_v, acc.at[dst_v.at[c]], ssems[b]).wait()

    def body(h, carry):
        c0 = h * NBUF
        for b in range(NBUF):
            c = c0 + b

            @pl.when(c >= NBUF)
            def _():
                scat_wait(c - NBUF, b)

            scat_start(c, b)
        return carry

    lax.fori_loop(0, CH // NBUF, body, 0)
    for b in range(NBUF):
        scat_wait(CH - NBUF + b, b)

    plsc.subcore_barrier()
    pltpu.sync_copy(acc.at[pl.ds(sid * SLAB, SLAB)],
                    dp_hbm.at[cid, pl.ds(sid * SLAB, SLAB)])


def _hop_ring(g_hbm, src_v, dst_v, rows_v, acc, gsems, ssems):
    def gather_start(c, b):
        pltpu.async_copy(g_hbm.at[src_v.at[c]], rows_v.at[b], gsems[b])

    def gather_wait(c, b):
        pltpu.make_async_copy(g_hbm.at[src_v.at[c]], rows_v.at[b],
                              gsems[b]).wait()

    def scat_start(c, b):
        pltpu.async_copy(rows_v.at[b], acc.at[dst_v.at[c]], ssems[b], add=True)

    def scat_wait(c, b):
        pltpu.make_async_copy(rows_v.at[b], acc.at[dst_v.at[c]],
                              ssems[b]).wait()

    for b in range(NBUF - 1):
        gather_start(b, b)

    def body(h, carry):
        c0 = h * NBUF
        for b in range(NBUF):
            c = c0 + b
            gather_wait(c, b)
            scat_start(c, b)
            bp = (b + NBUF - 1) % NBUF
            cp = c + NBUF - 1

            @pl.when(cp < CH)
            def _():
                @pl.when(c > 0)
                def _():
                    scat_wait(c - 1, bp)

                gather_start(cp, bp)
        return carry

    lax.fori_loop(0, CH // NBUF, body, 0)
    for b in range(NBUF):
        scat_wait(CH - NBUF + b, b)


def _fill_zbuf(zbuf):
    def fill(r, carry):
        for j in range(DH // 16):
            zbuf[r, pl.ds(j * 16, 16)] = jnp.zeros((16,), jnp.float32)
        return carry

    lax.fori_loop(0, ZR, fill, 0)


def _load_adjusted_idx(ei_hbm, src_v, dst_v, cid, sid):
    off = cid * N
    pltpu.sync_copy(ei_hbm.at[0, sid], src_v)
    pltpu.sync_copy(ei_hbm.at[1, sid], dst_v)

    def adj(c, carry):
        for j in range(K // 16):
            sl = pl.ds(j * 16, 16)
            src_v[c, sl] = src_v[c, sl] + off
        return carry

    lax.fori_loop(0, CH, adj, 0)


@functools.partial(
    pl.kernel,
    out_type=jax.ShapeDtypeStruct((NC, NP, DH), jnp.float32),
    mesh=_mesh,
    scratch_types=[
        pltpu.VMEM((CH, K), jnp.int32),
        pltpu.VMEM((CH, K), jnp.int32),
        pltpu.VMEM((NBUF, K, DH), jnp.float32),
        pltpu.VMEM((ZR, DH), jnp.float32),
        pltpu.VMEM_SHARED((NP, DH), jnp.float32),
        [pltpu.SemaphoreType.DMA] * NBUF,
        [pltpu.SemaphoreType.DMA] * NBUF,
    ],
    compiler_params=_sc_params,
)
def _hop_sc(g_hbm, ei_hbm, out_hbm, src_v, dst_v, rows_v, zbuf, acc,
            gsems, ssems):
    cid = lax.axis_index("c")
    sid = lax.axis_index("s")

    _fill_zbuf(zbuf)
    for z in range(SLAB // ZR):
        pltpu.sync_copy(zbuf, acc.at[pl.ds(sid * SLAB + z * ZR, ZR)])
    _load_adjusted_idx(ei_hbm, src_v, dst_v, cid, sid)
    plsc.subcore_barrier()

    _hop_ring(g_hbm, src_v, dst_v, rows_v, acc, gsems, ssems)

    plsc.subcore_barrier()
    pltpu.sync_copy(acc.at[pl.ds(sid * SLAB, SLAB)],
                    out_hbm.at[cid, pl.ds(sid * SLAB, SLAB)])


@functools.partial(
    pl.kernel,
    out_type=(jax.ShapeDtypeStruct((2 * N, DH), jnp.float32),
              jax.ShapeDtypeStruct((2 * N, DH), jnp.float32)),
    mesh=_mesh,
    scratch_types=[
        pltpu.VMEM((CH, K), jnp.int32),
        pltpu.VMEM((CH, K), jnp.int32),
        pltpu.VMEM((NBUF, K, DH), jnp.float32),
        pltpu.VMEM((ZR, DH), jnp.float32),
        pltpu.VMEM((SEG, 16), jnp.float32),
        pltpu.VMEM((SEG, DH), jnp.float32),
        pltpu.VMEM((SEG, DH), jnp.float32),
        pltpu.VMEM_SHARED((NP, DH), jnp.float32),
        [pltpu.SemaphoreType.DMA] * NBUF,
        [pltpu.SemaphoreType.DMA] * NBUF,
    ],
    compiler_params=_sc_params,
)
def _combhop_sc(dp_hbm, g0_hbm, p_hbm, ei_hbm, g1_hbm, h2_hbm, src_v, dst_v,
                rows_v, zbuf, degb, pb, g1b, acc, gsems, ssems):
    cid = lax.axis_index("c")
    sid = lax.axis_index("s")

    _fill_zbuf(zbuf)
    for z in range(SLAB // ZR):
        pltpu.sync_copy(zbuf, acc.at[pl.ds(sid * SLAB + z * ZR, ZR)])
    _load_adjusted_idx(ei_hbm, src_v, dst_v, cid, sid)

    for z in range(RSL // SEG):
        r0 = sid * RSL + z * SEG
        pltpu.sync_copy(dp_hbm.at[cid, pl.ds(r0, SEG)], degb)
        pltpu.sync_copy(p_hbm.at[cid, pl.ds(r0, SEG)], pb)
        pltpu.sync_copy(g0_hbm.at[pl.ds(cid * N + r0, SEG)], g1b)

        def comb(r, carry):
            rdeg = 1.0 / (degb[r, :] + 1.0)
            for j in range(DH // 16):
                sl = pl.ds(j * 16, 16)
                g1b[r, sl] = (pb[r, sl] + g1b[r, sl]) * rdeg
            return carry

        lax.fori_loop(0, SEG, comb, 0)
        pltpu.sync_copy(g1b, g1_hbm.at[pl.ds(cid * N + r0, SEG)])
    plsc.subcore_barrier()

    _hop_ring(g1_hbm, src_v, dst_v, rows_v, acc, gsems, ssems)
    plsc.subcore_barrier()

    for z in range(RSL // SEG):
        r0 = sid * RSL + z * SEG
        pltpu.sync_copy(acc.at[pl.ds(r0, SEG)], pb)
        pltpu.sync_copy(g1_hbm.at[pl.ds(cid * N + r0, SEG)], g1b)

        def fin(r, carry):
            for j in range(DH // 16):
                sl = pl.ds(j * 16, 16)
                pb[r, sl] = pb[r, sl] + g1b[r, sl]
            return carry

        lax.fori_loop(0, SEG, fin, 0)
        pltpu.sync_copy(pb, h2_hbm.at[pl.ds(cid * N + r0, SEG)])


_BR = 2000
_NB = N // _BR


def _prep_body(dp_ref, x_ref, o_ref):
    h = pl.program_id(0)
    deg = dp_ref[0, :, :1] + 1.0
    g = x_ref[...] * lax.rsqrt(deg)

    @pl.when(h == 0)
    def _():
        o_ref[...] = g[:, :DH]

    @pl.when(h == 1)
    def _():
        o_ref[...] = g[:, DH:]


_prep_tc = pl.pallas_call(
    _prep_body,
    grid=(2, _NB),
    in_specs=[pl.BlockSpec((NC, _BR, 16), lambda h, i: (0, i, 0)),
              pl.BlockSpec((_BR, D), lambda h, i: (i, 0))],
    out_specs=pl.BlockSpec((_BR, DH), lambda h, i: (h * _NB + i, 0)),
    out_shape=jax.ShapeDtypeStruct((2 * N, DH), jnp.float32),
)


def _final_body(dp_ref, glo_ref, ghi_ref, w1_ref, b1_ref, gm_ref, bt_ref,
                w2_ref, b2_ref, o_ref):
    deg = dp_ref[0, :, :1] + 1.0
    h2 = jnp.concatenate([glo_ref[...], ghi_ref[...]], axis=1)
    h2 = h2 * lax.rsqrt(deg)
    t = lax.dot_general(h2, w1_ref[...], (((1,), (1,)), ((), ())),
                        preferred_element_type=jnp.float32)
    bn_scale = np.float32(1.0 / np.sqrt(1.0 + BN_EPS))
    t = (t + b1_ref[...]) * (gm_ref[...] * bn_scale) + bt_ref[...]
    t = jnp.maximum(t, 0.0)
    o_ref[...] = lax.dot_general(t, w2_ref[...], (((1,), (1,)), ((), ())),
                                 preferred_element_type=jnp.float32) + b2_ref[...]


def _full_spec(shape):
    nd = len(shape)
    return pl.BlockSpec(shape, lambda i, _nd=nd: (0,) * _nd)


_final_tc = pl.pallas_call(
    _final_body,
    grid=(_NB,),
    in_specs=[pl.BlockSpec((NC, _BR, 16), lambda i: (0, i, 0)),
              pl.BlockSpec((_BR, DH), lambda i: (i, 0)),
              pl.BlockSpec((_BR, DH), lambda i: (_NB + i, 0)),
              _full_spec((D, D)), _full_spec((1, D)), _full_spec((1, D)),
              _full_spec((1, D)), _full_spec((D, D)), _full_spec((1, D))],
    out_specs=pl.BlockSpec((_BR, D), lambda i: (i, 0)),
    out_shape=jax.ShapeDtypeStruct((N, D), jnp.float32),
)


def kernel(x, edge_index, W1, b1, gamma, beta, W2, b2):
    ei = edge_index.astype(jnp.int32).reshape(2, NS, CH, K)
    b1r = b1.reshape(1, D)
    gmr = gamma.reshape(1, D)
    btr = beta.reshape(1, D)
    b2r = b2.reshape(1, D)
    dp = _deg_sc(ei)
    g0 = _prep_tc(dp, x)
    p = _hop_sc(g0, ei)
    g1, h2u = _combhop_sc(dp, g0, p, ei)
    del g1
    out = _final_tc(dp, h2u, h2u, W1, b1r, gmr, btr, W2, b2r)
    return out


# speedup vs baseline: 33.9014x; 1.0268x over previous
import functools

import jax
import jax.numpy as jnp
import numpy as np
from jax import lax
from jax.experimental import pallas as pl
from jax.experimental.pallas import tpu as pltpu
from jax.experimental.pallas import tpu_sc as plsc

N = 10000
NP = 10240
E = 320000
D = 128
DH = D // 2
BN_EPS = 1e-5

NC = 2
NS = 16

K = 80
CH = (E // NS) // K
NBUF = 5
ZR = 64
SLAB = NP // NS
RSL = N // NS
SEG = 125

_mesh = plsc.VectorSubcoreMesh(core_axis_name="c", subcore_axis_name="s")
_sc_params = pltpu.CompilerParams(use_tc_tiling_on_sc=False)


@functools.partial(
    pl.kernel,
    out_type=jax.ShapeDtypeStruct((NC, NP, 16), jnp.float32),
    mesh=_mesh,
    scratch_types=[
        pltpu.VMEM((CH, K), jnp.int32),
        pltpu.VMEM((K, 16), jnp.float32),
        pltpu.VMEM((ZR, 16), jnp.float32),
        pltpu.VMEM_SHARED((NP, 16), jnp.float32),
        [pltpu.SemaphoreType.DMA] * NBUF,
    ],
    compiler_params=_sc_params,
)
def _deg_sc(ei_hbm, dp_hbm, dst_v, ones_v, zbuf, acc, ssems):
    cid = lax.axis_index("c")
    sid = lax.axis_index("s")

    def fill(r, carry):
        zbuf[r, :] = jnp.zeros((16,), jnp.float32)
        return carry

    lax.fori_loop(0, ZR, fill, 0)

    def fill1(r, carry):
        ones_v[r, :] = jnp.ones((16,), jnp.float32)
        return carry

    lax.fori_loop(0, K, fill1, 0)

    for z in range(SLAB // ZR):
        pltpu.sync_copy(zbuf, acc.at[pl.ds(sid * SLAB + z * ZR, ZR)])
    plsc.subcore_barrier()

    pltpu.sync_copy(ei_hbm.at[1, sid], dst_v)

    def scat_start(c, b):
        pltpu.async_copy(ones_v, acc.at[dst_v.at[c]], ssems[b], add=True)

    def scat_wait(c, b):
        pltpu.make_async_copy(ones_v, acc.at[dst_v.at[c]], ssems[b]).wait()

    def body(h, carry):
        c0 = h * NBUF
        for b in range(NBUF):
            c = c0 + b

            @pl.when(c >= NBUF)
            def _():
                scat_wait(c - NBUF, b)

            scat_start(c, b)
        return carry

    lax.fori_loop(0, CH // NBUF, body, 0)
    for b in range(NBUF):
        scat_wait(CH - NBUF + b, b)

    plsc.subcore_barrier()
    pltpu.sync_copy(acc.at[pl.ds(sid * SLAB, SLAB)],
                    dp_hbm.at[cid, pl.ds(sid * SLAB, SLAB)])


def _hop_ring(g_hbm, src_v, dst_v, rows_v, acc, gsems, ssems):
    def gather_start(c, b):
        pltpu.async_copy(g_hbm.at[src_v.at[c]], rows_v.at[b], gsems[b])

    def gather_wait(c, b):
        pltpu.make_async_copy(g_hbm.at[src_v.at[c]], rows_v.at[b],
                              gsems[b]).wait()

    def scat_start(c, b):
        pltpu.async_copy(rows_v.at[b], acc.at[dst_v.at[c]], ssems[b], add=True)

    def scat_wait(c, b):
        pltpu.make_async_copy(rows_v.at[b], acc.at[dst_v.at[c]],
                              ssems[b]).wait()

    for b in range(NBUF - 1):
        gather_start(b, b)

    def body(h, carry):
        c0 = h * NBUF
        for b in range(NBUF):
            c = c0 + b
            gather_wait(c, b)
            scat_start(c, b)
            bp = (b + NBUF - 1) % NBUF
            cp = c + NBUF - 1

            @pl.when(cp < CH)
            def _():
                @pl.when(c > 0)
                def _():
                    scat_wait(c - 1, bp)

                gather_start(cp, bp)
        return carry

    lax.fori_loop(0, CH // NBUF, body, 0)
    for b in range(NBUF):
        scat_wait(CH - NBUF + b, b)


def _fill_zbuf(zbuf):
    def fill(r, carry):
        for j in range(DH // 16):
            zbuf[r, pl.ds(j * 16, 16)] = jnp.zeros((16,), jnp.float32)
        return carry

    lax.fori_loop(0, ZR, fill, 0)


def _load_adjusted_idx(ei_hbm, src_v, dst_v, cid, sid):
    off = cid * N
    pltpu.sync_copy(ei_hbm.at[0, sid], src_v)
    pltpu.sync_copy(ei_hbm.at[1, sid], dst_v)

    def adj(c, carry):
        for j in range(K // 16):
            sl = pl.ds(j * 16, 16)
            src_v[c, sl] = src_v[c, sl] + off
        return carry

    lax.fori_loop(0, CH, adj, 0)


@functools.partial(
    pl.kernel,
    out_type=jax.ShapeDtypeStruct((NC, NP, DH), jnp.float32),
    mesh=_mesh,
    scratch_types=[
        pltpu.VMEM((CH, K), jnp.int32),
        pltpu.VMEM((CH, K), jnp.int32),
        pltpu.VMEM((NBUF, K, DH), jnp.float32),
        pltpu.VMEM((ZR, DH), jnp.float32),
        pltpu.VMEM_SHARED((NP, DH), jnp.float32),
        [pltpu.SemaphoreType.DMA] * NBUF,
        [pltpu.SemaphoreType.DMA] * NBUF,
    ],
    compiler_params=_sc_params,
)
def _hop_sc(g_hbm, ei_hbm, out_hbm, src_v, dst_v, rows_v, zbuf, acc,
            gsems, ssems):
    cid = lax.axis_index("c")
    sid = lax.axis_index("s")

    _fill_zbuf(zbuf)
    for z in range(SLAB // ZR):
        pltpu.sync_copy(zbuf, acc.at[pl.ds(sid * SLAB + z * ZR, ZR)])
    _load_adjusted_idx(ei_hbm, src_v, dst_v, cid, sid)
    plsc.subcore_barrier()

    _hop_ring(g_hbm, src_v, dst_v, rows_v, acc, gsems, ssems)

    plsc.subcore_barrier()
    pltpu.sync_copy(acc.at[pl.ds(sid * SLAB, SLAB)],
                    out_hbm.at[cid, pl.ds(sid * SLAB, SLAB)])


@functools.partial(
    pl.kernel,
    out_type=(jax.ShapeDtypeStruct((2 * N, DH), jnp.float32),
              jax.ShapeDtypeStruct((2 * N, DH), jnp.float32)),
    mesh=_mesh,
    scratch_types=[
        pltpu.VMEM((CH, K), jnp.int32),
        pltpu.VMEM((CH, K), jnp.int32),
        pltpu.VMEM((NBUF, K, DH), jnp.float32),
        pltpu.VMEM((ZR, DH), jnp.float32),
        pltpu.VMEM((SEG, 16), jnp.float32),
        pltpu.VMEM((SEG, DH), jnp.float32),
        pltpu.VMEM((SEG, DH), jnp.float32),
        pltpu.VMEM_SHARED((NP, DH), jnp.float32),
        [pltpu.SemaphoreType.DMA] * NBUF,
        [pltpu.SemaphoreType.DMA] * NBUF,
    ],
    compiler_params=_sc_params,
)
def _combhop_sc(dp_hbm, g0_hbm, p_hbm, ei_hbm, g1_hbm, h2_hbm, src_v, dst_v,
                rows_v, zbuf, degb, pb, g1b, acc, gsems, ssems):
    cid = lax.axis_index("c")
    sid = lax.axis_index("s")

    _fill_zbuf(zbuf)
    for z in range(SLAB // ZR):
        pltpu.sync_copy(zbuf, acc.at[pl.ds(sid * SLAB + z * ZR, ZR)])
    _load_adjusted_idx(ei_hbm, src_v, dst_v, cid, sid)

    for z in range(RSL // SEG):
        r0 = sid * RSL + z * SEG
        d0 = pltpu.async_copy(dp_hbm.at[cid, pl.ds(r0, SEG)], degb, gsems[0])
        d1 = pltpu.async_copy(p_hbm.at[cid, pl.ds(r0, SEG)], pb, gsems[1])
        d2 = pltpu.async_copy(g0_hbm.at[pl.ds(cid * N + r0, SEG)], g1b,
                              gsems[2])
        d0.wait()
        d1.wait()
        d2.wait()

        def comb(r, carry):
            rdeg = 1.0 / (degb[r, :] + 1.0)
            for j in range(DH // 16):
                sl = pl.ds(j * 16, 16)
                g1b[r, sl] = (pb[r, sl] + g1b[r, sl]) * rdeg
            return carry

        lax.fori_loop(0, SEG, comb, 0)
        pltpu.sync_copy(g1b, g1_hbm.at[pl.ds(cid * N + r0, SEG)])
    plsc.subcore_barrier()

    _hop_ring(g1_hbm, src_v, dst_v, rows_v, acc, gsems, ssems)
    plsc.subcore_barrier()

    for z in range(RSL // SEG):
        r0 = sid * RSL + z * SEG
        d0 = pltpu.async_copy(acc.at[pl.ds(r0, SEG)], pb, gsems[0])
        d1 = pltpu.async_copy(g1_hbm.at[pl.ds(cid * N + r0, SEG)], g1b,
                              gsems[1])
        d0.wait()
        d1.wait()

        def fin(r, carry):
            for j in range(DH // 16):
                sl = pl.ds(j * 16, 16)
                pb[r, sl] = pb[r, sl] + g1b[r, sl]
            return carry

        lax.fori_loop(0, SEG, fin, 0)
        pltpu.sync_copy(pb, h2_hbm.at[pl.ds(cid * N + r0, SEG)])


_BR = 2000
_NB = N // _BR


def _prep_body(dp_ref, x_ref, o_ref):
    h = pl.program_id(0)
    deg = dp_ref[0, :, :1] + 1.0
    g = x_ref[...] * lax.rsqrt(deg)

    @pl.when(h == 0)
    def _():
        o_ref[...] = g[:, :DH]

    @pl.when(h == 1)
    def _():
        o_ref[...] = g[:, DH:]


_prep_tc = pl.pallas_call(
    _prep_body,
    grid=(2, _NB),
    in_specs=[pl.BlockSpec((NC, _BR, 16), lambda h, i: (0, i, 0)),
              pl.BlockSpec((_BR, D), lambda h, i: (i, 0))],
    out_specs=pl.BlockSpec((_BR, DH), lambda h, i: (h * _NB + i, 0)),
    out_shape=jax.ShapeDtypeStruct((2 * N, DH), jnp.float32),
)


def _final_body(dp_ref, glo_ref, ghi_ref, w1_ref, b1_ref, gm_ref, bt_ref,
                w2_ref, b2_ref, o_ref):
    deg = dp_ref[0, :, :1] + 1.0
    h2 = jnp.concatenate([glo_ref[...], ghi_ref[...]], axis=1)
    h2 = h2 * lax.rsqrt(deg)
    t = lax.dot_general(h2, w1_ref[...], (((1,), (1,)), ((), ())),
                        preferred_element_type=jnp.float32)
    bn_scale = np.float32(1.0 / np.sqrt(1.0 + BN_EPS))
    t = (t + b1_ref[...]) * (gm_ref[...] * bn_scale) + bt_ref[...]
    t = jnp.maximum(t, 0.0)
    o_ref[...] = lax.dot_general(t, w2_ref[...], (((1,), (1,)), ((), ())),
                                 preferred_element_type=jnp.float32) + b2_ref[...]


def _full_spec(shape):
    nd = len(shape)
    return pl.BlockSpec(shape, lambda i, _nd=nd: (0,) * _nd)


_final_tc = pl.pallas_call(
    _final_body,
    grid=(_NB,),
    in_specs=[pl.BlockSpec((NC, _BR, 16), lambda i: (0, i, 0)),
              pl.BlockSpec((_BR, DH), lambda i: (i, 0)),
              pl.BlockSpec((_BR, DH), lambda i: (_NB + i, 0)),
              _full_spec((D, D)), _full_spec((1, D)), _full_spec((1, D)),
              _full_spec((1, D)), _full_spec((D, D)), _full_spec((1, D))],
    out_specs=pl.BlockSpec((_BR, D), lambda i: (i, 0)),
    out_shape=jax.ShapeDtypeStruct((N, D), jnp.float32),
)


def kernel(x, edge_index, W1, b1, gamma, beta, W2, b2):
    ei = edge_index.astype(jnp.int32).reshape(2, NS, CH, K)
    b1r = b1.reshape(1, D)
    gmr = gamma.reshape(1, D)
    btr = beta.reshape(1, D)
    b2r = b2.reshape(1, D)
    dp = _deg_sc(ei)
    g0 = _prep_tc(dp, x)
    p = _hop_sc(g0, ei)
    g1, h2u = _combhop_sc(dp, g0, p, ei)
    del g1
    out = _final_tc(dp, h2u, h2u, W1, b1r, gmr, btr, W2, b2r)
    return out
